# baseline (speedup 1.0000x reference)
import functools

import jax
import jax.numpy as jnp
from jax import lax
from jax.experimental import pallas as pl
from jax.experimental.pallas import tpu as pltpu
from jax.experimental.pallas import tpu_sc as plsc

N_ID = 13
N_DENSE = 13
EMB = 32
BATCH = 16384
VOCAB = 2000
NG = 4
PER_G = 4


def _make_sc_gather():
    info = plsc.get_sparse_core_info()
    nw = info.num_cores * info.num_subcores
    per_k = nw // NG
    rows_w = BATCH // per_k
    n_chunks = 8
    chunk = rows_w // n_chunks

    mesh = plsc.VectorSubcoreMesh(core_axis_name="c", subcore_axis_name="s")

    @functools.partial(
        pl.kernel,
        mesh=mesh,
        out_type=jax.ShapeDtypeStruct((NG, BATCH, PER_G * EMB), jnp.float32),
        scratch_types=[
            pltpu.VMEM((PER_G, rows_w), jnp.int32),
            pltpu.VMEM((2, PER_G, chunk, EMB), jnp.float32),
            pltpu.SemaphoreType.DMA,
            pltpu.SemaphoreType.DMA,
            pltpu.SemaphoreType.DMA,
            pltpu.SemaphoreType.DMA,
        ],
        compiler_params=pltpu.CompilerParams(use_tc_tiling_on_sc=False,
                                             needs_layout_passes=False),
    )
    def gather_k(table_hbm, idx_hbm, out_hbm, idx_v, rows_v, g0, g1, w0, w1):
        wid = lax.axis_index("s") * info.num_cores + lax.axis_index("c")
        k = wid // per_k
        sub = wid % per_k
        base = sub * rows_w
        gsem = [g0, g1]
        wsem = [w0, w1]

        for t in range(PER_G):
            pltpu.sync_copy(
                idx_hbm.at[pl.ds((k * PER_G + t) * BATCH + base, rows_w)],
                idx_v.at[t])

        def fire(c, slot):
            cps = []
            for t in range(PER_G):
                cps.append(pltpu.async_copy(
                    table_hbm.at[idx_v.at[t, pl.ds(c * chunk, chunk)]],
                    rows_v.at[slot, t], gsem[slot]))
            return cps

        def writeback(c, slot):
            roff = base + c * chunk
            cps = []
            for t in range(PER_G):
                cps.append(pltpu.async_copy(
                    rows_v.at[slot, t],
                    out_hbm.at[k, pl.ds(roff, chunk),
                               pl.ds(t * EMB, EMB)],
                    wsem[slot]))
            return cps

        gps = [fire(0, 0), None]
        wps = [None, None]
        for c in range(n_chunks):
            slot = c % 2
            nxt = 1 - slot
            if c + 1 < n_chunks:
                if wps[nxt] is not None:
                    for cp in wps[nxt]:
                        cp.wait()
                    wps[nxt] = None
                gps[nxt] = fire(c + 1, nxt)
            for cp in gps[slot]:
                cp.wait()
            wps[slot] = writeback(c, slot)
        for side in wps:
            if side is not None:
                for cp in side:
                    cp.wait()

    return gather_k


@functools.lru_cache(maxsize=None)
def _sc_gather_cached():
    return _make_sc_gather()


def _mlp_body(e_ref, d_ref, w1e_ref, w1d_ref, b1_ref, w2_ref, b2_ref,
              w3_ref, b3_ref, out_ref):
    x = jnp.dot(d_ref[...], w1d_ref[...], preferred_element_type=jnp.float32)
    for k in range(NG):
        x += jnp.dot(e_ref[k].astype(jnp.bfloat16), w1e_ref[k],
                     preferred_element_type=jnp.float32)
    x += b1_ref[...]
    h = jnp.maximum(x, 0.0).astype(jnp.bfloat16)
    h = jnp.maximum(
        jnp.dot(h, w2_ref[...], preferred_element_type=jnp.float32)
        + b2_ref[...], 0.0).astype(jnp.bfloat16)
    out_ref[...] = (
        jnp.dot(h, w3_ref[...], preferred_element_type=jnp.float32)
        + b3_ref[...])


def _mlp(e3, d, w1e, w1d, b1, w2, b2, w3, b3, tb: int = 1024):
    grid = (BATCH // tb,)
    full2 = lambda shape: pl.BlockSpec(shape, lambda i: (0, 0))
    full3 = lambda shape: pl.BlockSpec(shape, lambda i: (0, 0, 0))
    return pl.pallas_call(
        _mlp_body,
        grid=grid,
        in_specs=[
            pl.BlockSpec((NG, tb, PER_G * EMB), lambda i: (0, i, 0)),
            pl.BlockSpec((tb, N_DENSE), lambda i: (i, 0)),
            full3(w1e.shape),
            full2(w1d.shape),
            full2(b1.shape),
            full2(w2.shape),
            full2(b2.shape),
            full2(w3.shape),
            full2(b3.shape),
        ],
        out_specs=pl.BlockSpec((tb, 1), lambda i: (i, 0)),
        out_shape=jax.ShapeDtypeStruct((BATCH, 1), jnp.float32),
    )(e3, d, w1e, w1d, b1, w2, b2, w3, b3)


def kernel(id_feature, dense_feature, base_embedding, W1, b1, W2, b2, W3, b3):
    bf = jnp.bfloat16
    ids16 = jnp.pad(id_feature[:, :N_ID].astype(jnp.int32),
                    ((0, 0), (0, NG * PER_G - N_ID)))
    ids_r = ids16.T.reshape(-1)
    d = dense_feature[:, -N_DENSE:].astype(bf)
    ew = N_ID * EMB
    w1a = W1[:ew] + W1[ew:2 * ew]
    w1d = (W1[2 * ew:2 * ew + N_DENSE] + W1[2 * ew + N_DENSE:]).astype(bf)
    w1e = jnp.pad(w1a, ((0, NG * PER_G * EMB - ew), (0, 0))).astype(bf)
    w1e = w1e.reshape(NG, PER_G * EMB, -1)

    e3 = _sc_gather_cached()(base_embedding, ids_r)

    return _mlp(e3, d, w1e, w1d,
                b1.reshape(1, -1), W2.astype(bf), b2.reshape(1, -1),
                W3.astype(bf), b3.reshape(1, -1))


# speedup vs baseline: 1.0265x; 1.0265x over previous
import functools

import jax
import jax.numpy as jnp
from jax import lax
from jax.experimental import pallas as pl
from jax.experimental.pallas import tpu as pltpu
from jax.experimental.pallas import tpu_sc as plsc

N_ID = 13
N_DENSE = 13
EMB = 32
BATCH = 16384
VOCAB = 2000
NG = 4
PER_G = 4


def _make_sc_gather():
    info = plsc.get_sparse_core_info()
    nw = info.num_cores * info.num_subcores
    per_k = nw // NG
    rows_w = BATCH // per_k
    n_chunks = 16
    chunk = rows_w // n_chunks

    mesh = plsc.VectorSubcoreMesh(core_axis_name="c", subcore_axis_name="s")

    @functools.partial(
        pl.kernel,
        mesh=mesh,
        out_type=jax.ShapeDtypeStruct((NG, BATCH, PER_G * EMB), jnp.float32),
        scratch_types=[
            pltpu.VMEM((PER_G, rows_w), jnp.int32),
            pltpu.VMEM((2, PER_G, chunk, EMB), jnp.float32),
            pltpu.VMEM_SHARED((info.num_subcores, 2, chunk, PER_G * EMB),
                              jnp.float32),
            pltpu.SemaphoreType.DMA,
            pltpu.SemaphoreType.DMA,
            pltpu.SemaphoreType.DMA,
            pltpu.SemaphoreType.DMA,
        ],
        compiler_params=pltpu.CompilerParams(use_tc_tiling_on_sc=False,
                                             needs_layout_passes=False),
    )
    def gather_k(table_hbm, idx_hbm, out_hbm, idx_v, rows_v, ilv_v,
                 g0, g1, w0, w1):
        sidx = lax.axis_index("s")
        wid = sidx * info.num_cores + lax.axis_index("c")
        k = wid // per_k
        sub = wid % per_k
        base = sub * rows_w
        gsem = [g0, g1]
        wsem = [w0, w1]

        for t in range(PER_G):
            pltpu.sync_copy(
                idx_hbm.at[pl.ds((k * PER_G + t) * BATCH + base, rows_w)],
                idx_v.at[t])

        def fire(c, slot):
            return [pltpu.async_copy(
                table_hbm.at[idx_v.at[t, pl.ds(c * chunk, chunk)]],
                rows_v.at[slot, t], gsem[slot]) for t in range(PER_G)]

        gps = [fire(0, 0), None]
        wps = [None, None]
        for c in range(n_chunks):
            slot = c % 2
            nxt = 1 - slot
            if c + 1 < n_chunks:
                if wps[nxt] is not None:
                    for cp in wps[nxt]:
                        cp.wait()
                    wps[nxt] = None
                gps[nxt] = fire(c + 1, nxt)
            for cp in gps[slot]:
                cp.wait()
            for t in range(PER_G):
                pltpu.sync_copy(rows_v.at[slot, t],
                                ilv_v.at[sidx, slot, :, pl.ds(t * EMB, EMB)])
            wps[slot] = [pltpu.async_copy(
                ilv_v.at[sidx, slot],
                out_hbm.at[k, pl.ds(base + c * chunk, chunk), :],
                wsem[slot])]
        for side in wps:
            if side is not None:
                for cp in side:
                    cp.wait()

    return gather_k


@functools.lru_cache(maxsize=None)
def _sc_gather_cached():
    return _make_sc_gather()


def _mlp_body(e_ref, d_ref, w1e_ref, w1d_ref, b1_ref, w2_ref, b2_ref,
              w3_ref, b3_ref, out_ref):
    x = jnp.dot(d_ref[...], w1d_ref[...], preferred_element_type=jnp.float32)
    for k in range(NG):
        x += jnp.dot(e_ref[k].astype(jnp.bfloat16), w1e_ref[k],
                     preferred_element_type=jnp.float32)
    x += b1_ref[...]
    h = jnp.maximum(x, 0.0).astype(jnp.bfloat16)
    h = jnp.maximum(
        jnp.dot(h, w2_ref[...], preferred_element_type=jnp.float32)
        + b2_ref[...], 0.0).astype(jnp.bfloat16)
    out_ref[...] = (
        jnp.dot(h, w3_ref[...], preferred_element_type=jnp.float32)
        + b3_ref[...])


def _mlp(e3, d, w1e, w1d, b1, w2, b2, w3, b3, tb: int = 1024):
    grid = (BATCH // tb,)
    full2 = lambda shape: pl.BlockSpec(shape, lambda i: (0, 0))
    full3 = lambda shape: pl.BlockSpec(shape, lambda i: (0, 0, 0))
    return pl.pallas_call(
        _mlp_body,
        grid=grid,
        in_specs=[
            pl.BlockSpec((NG, tb, PER_G * EMB), lambda i: (0, i, 0)),
            pl.BlockSpec((tb, N_DENSE), lambda i: (i, 0)),
            full3(w1e.shape),
            full2(w1d.shape),
            full2(b1.shape),
            full2(w2.shape),
            full2(b2.shape),
            full2(w3.shape),
            full2(b3.shape),
        ],
        out_specs=pl.BlockSpec((tb, 1), lambda i: (i, 0)),
        out_shape=jax.ShapeDtypeStruct((BATCH, 1), jnp.float32),
    )(e3, d, w1e, w1d, b1, w2, b2, w3, b3)


def kernel(id_feature, dense_feature, base_embedding, W1, b1, W2, b2, W3, b3):
    bf = jnp.bfloat16
    ids16 = jnp.pad(id_feature[:, :N_ID].astype(jnp.int32),
                    ((0, 0), (0, NG * PER_G - N_ID)))
    ids_r = ids16.T.reshape(-1)
    d = dense_feature[:, -N_DENSE:].astype(bf)
    ew = N_ID * EMB
    w1a = W1[:ew] + W1[ew:2 * ew]
    w1d = (W1[2 * ew:2 * ew + N_DENSE] + W1[2 * ew + N_DENSE:]).astype(bf)
    w1e = jnp.pad(w1a, ((0, NG * PER_G * EMB - ew), (0, 0))).astype(bf)
    w1e = w1e.reshape(NG, PER_G * EMB, -1)

    e3 = _sc_gather_cached()(base_embedding, ids_r)

    return _mlp(e3, d, w1e, w1d,
                b1.reshape(1, -1), W2.astype(bf), b2.reshape(1, -1),
                W3.astype(bf), b3.reshape(1, -1))
